# baseline (speedup 1.0000x reference)
import functools

import jax
import jax.numpy as jnp
from jax import lax
from jax.experimental import pallas as pl
from jax.experimental.pallas import tpu as pltpu

B = 16384
D = 128
NREL = 1000
NRELP = 1024
BLK = 512
NB = B // BLK


def _tc_body(rel_ref, r_ref, h_ref, t_ref, out_ref):
    r2 = r_ref[0]
    k_iota = lax.broadcasted_iota(jnp.int32, (BLK, NRELP), 1)
    oh = (k_iota == r2).astype(jnp.bfloat16)
    g = lax.dot_general(oh, rel_ref[...], (((1,), (0,)), ((), ())),
                        preferred_element_type=jnp.float32)
    s = jnp.sum(h_ref[...] * g * t_ref[...], axis=1, keepdims=True)
    out_ref[...] = s


_tc_call = pl.pallas_call(
    _tc_body,
    grid=(NB,),
    in_specs=[
        pl.BlockSpec((NRELP, D), lambda i: (0, 0)),
        pl.BlockSpec((1, BLK, 1), lambda i: (i, 0, 0)),
        pl.BlockSpec((BLK, D), lambda i: (i, 0)),
        pl.BlockSpec((BLK, D), lambda i: (i, 0)),
    ],
    out_specs=pl.BlockSpec((BLK, 1), lambda i: (i, 0)),
    out_shape=jax.ShapeDtypeStruct((B, 1), jnp.float32),
)


def kernel(h, r, t, mode, rel_emb):
    del mode
    r32 = r.astype(jnp.int32).reshape(NB, BLK, 1)
    relp = jnp.zeros((NRELP, D), jnp.bfloat16).at[:NREL].set(
        rel_emb.astype(jnp.bfloat16))
    return _tc_call(relp, r32, h, t).reshape(B)
